# baseline (speedup 1.0000x reference)
import functools

import jax
import jax.numpy as jnp
from jax import lax
from jax.experimental import pallas as pl
from jax.experimental.pallas import tpu as pltpu
from jax.experimental.pallas import tpu_sc as plsc

N = 16384
C = 1000
S = 8
GP = 8192

COLS = 1024
GRID = N // COLS

NSUB = 16
PERH = N // NSUB
BPW = GP // NSUB
HCH = PERH // 128


def _sc_w_body(t_hbm, s_hbm, w_hbm,
               t_v, s_v, gid2d, ones_v, zeros_v, cnt_v, w_v, sem,
               shared_cnt):
    cid = lax.axis_index("c")
    sid = lax.axis_index("s")

    one16 = jnp.full((16,), 1.0, jnp.float32)
    zero16 = jnp.zeros((16,), jnp.float32)

    hbase = sid * PERH
    pltpu.sync_copy(t_hbm.at[pl.ds(hbase, PERH)], t_v)
    pltpu.sync_copy(s_hbm.at[pl.ds(hbase, PERH)], s_v)

    def _fill_ones(i, _):
        ones_v[pl.ds(i * 16, 16)] = one16
        return 0

    def _fill_zeros(i, _):
        zeros_v[pl.ds(i * 16, 16)] = zero16
        return 0

    def _gid(k, _):
        t16 = t_v[pl.ds(k * 16, 16)]
        s16 = s_v[pl.ds(k * 16, 16)]
        gid2d[k // 8, pl.ds((k % 8) * 16, 16)] = t16 * S + s16
        return 0

    lax.fori_loop(0, 8, _fill_ones, 0, unroll=False)
    lax.fori_loop(0, BPW // 16, _fill_zeros, 0, unroll=False)
    pltpu.sync_copy(zeros_v, shared_cnt.at[pl.ds(sid * BPW, BPW)])
    lax.fori_loop(0, PERH // 16, _gid, 0, unroll=False)

    plsc.subcore_barrier()

    def _scat(j, _):
        pltpu.sync_copy(ones_v, shared_cnt.at[gid2d.at[j]], add=True)
        return 0

    lax.fori_loop(0, HCH, _scat, 0, unroll=False)
    plsc.subcore_barrier()

    def _gath(j, _):
        pltpu.sync_copy(shared_cnt.at[gid2d.at[cid * 4 + j]], cnt_v.at[j])
        return 0

    def _recip(k, _):
        c16 = cnt_v[k // 8, pl.ds((k % 8) * 16, 16)]
        w_v[pl.ds(k * 16, 16)] = 1.0 / c16
        return 0

    lax.fori_loop(0, 4, _gath, 0, unroll=False)
    lax.fori_loop(0, 32, _recip, 0, unroll=False)
    pltpu.sync_copy(w_v, w_hbm.at[pl.ds(hbase + cid * 512, 512)])


def _sc_weights(targets, subgroup_inf):
    mesh = plsc.VectorSubcoreMesh(core_axis_name="c", subcore_axis_name="s")
    f = functools.partial(
        pl.kernel,
        out_type=jax.ShapeDtypeStruct((N,), jnp.float32),
        mesh=mesh,
        scratch_types=[
            pltpu.VMEM((PERH,), jnp.int32),
            pltpu.VMEM((PERH,), jnp.int32),
            pltpu.VMEM((HCH, 128), jnp.int32),
            pltpu.VMEM((128,), jnp.float32),
            pltpu.VMEM((BPW,), jnp.float32),
            pltpu.VMEM((4, 128), jnp.float32),
            pltpu.VMEM((512,), jnp.float32),
            pltpu.SemaphoreType.DMA,
            pltpu.VMEM_SHARED((GP,), jnp.float32),
        ],
    )(_sc_w_body)
    return f(targets, subgroup_inf)


def _tc_body(x_ref, t_ref, o_ref):
    x = x_ref[...]
    t = t_ref[...]
    ones = jnp.ones((1, C), jnp.float32)
    m = jnp.max(x, axis=0, keepdims=True)
    e = jnp.exp(x - m)
    row = lax.broadcasted_iota(jnp.int32, (C, COLS), 0)
    px = jnp.where(row == t, x, 0.0)
    ssum = jnp.dot(ones, e, preferred_element_type=jnp.float32)
    picked = jnp.dot(ones, px, preferred_element_type=jnp.float32)
    o_ref[...] = m + jnp.log(ssum) - picked


def _tc_loss(logits_t, targets):
    return pl.pallas_call(
        _tc_body,
        grid=(GRID,),
        in_specs=[
            pl.BlockSpec((C, COLS), lambda i: (0, i)),
            pl.BlockSpec((1, COLS), lambda i: (0, i)),
        ],
        out_specs=pl.BlockSpec((1, COLS), lambda i: (0, i)),
        out_shape=jax.ShapeDtypeStruct((1, N), jnp.float32),
    )(logits_t, targets[None, :])


def _tc_dot_body(l_ref, w_ref, o_ref):
    o_ref[...] = jnp.sum(l_ref[...] * w_ref[...]).reshape(1, 1)


def _tc_dot(loss, w):
    return pl.pallas_call(
        _tc_dot_body,
        in_specs=[
            pl.BlockSpec((1, N), lambda: (0, 0)),
            pl.BlockSpec((1, N), lambda: (0, 0)),
        ],
        out_specs=pl.BlockSpec((1, 1), lambda: (0, 0)),
        out_shape=jax.ShapeDtypeStruct((1, 1), jnp.float32),
    )(loss, w[None, :])


def kernel(logits, targets, subgroup_inf):
    w = _sc_weights(targets, subgroup_inf)
    loss = _tc_loss(logits.T, targets)
    tot = _tc_dot(loss, w)
    return tot.reshape(())


# speedup vs baseline: 1.0408x; 1.0408x over previous
import functools

import jax
import jax.numpy as jnp
from jax import lax
from jax.experimental import pallas as pl
from jax.experimental.pallas import tpu as pltpu
from jax.experimental.pallas import tpu_sc as plsc

N = 16384
C = 1000
S = 8
GP = 8192

CROWS = 200
CGRID = C // CROWS

NSUB = 16
PERH = N // NSUB
BPW = GP // NSUB
HCH = PERH // 128


def _sc_w_body(t_hbm, s_hbm, w_hbm,
               t_v, s_v, gid2d, ones_v, zeros_v, cnt_v, w_v, sem,
               shared_cnt):
    cid = lax.axis_index("c")
    sid = lax.axis_index("s")

    one16 = jnp.full((16,), 1.0, jnp.float32)
    zero16 = jnp.zeros((16,), jnp.float32)

    hbase = sid * PERH
    pltpu.sync_copy(t_hbm.at[pl.ds(hbase, PERH)], t_v)
    pltpu.sync_copy(s_hbm.at[pl.ds(hbase, PERH)], s_v)

    def _fill_ones(i, _):
        ones_v[pl.ds(i * 16, 16)] = one16
        return 0

    def _fill_zeros(i, _):
        zeros_v[pl.ds(i * 16, 16)] = zero16
        return 0

    def _gid(k, _):
        t16 = t_v[pl.ds(k * 16, 16)]
        s16 = s_v[pl.ds(k * 16, 16)]
        gid2d[k // 8, pl.ds((k % 8) * 16, 16)] = t16 * S + s16
        return 0

    lax.fori_loop(0, 8, _fill_ones, 0, unroll=False)
    lax.fori_loop(0, BPW // 16, _fill_zeros, 0, unroll=False)
    pltpu.sync_copy(zeros_v, shared_cnt.at[pl.ds(sid * BPW, BPW)])
    lax.fori_loop(0, PERH // 16, _gid, 0, unroll=False)

    plsc.subcore_barrier()

    def _scat(j, _):
        pltpu.sync_copy(ones_v, shared_cnt.at[gid2d.at[j]], add=True)
        return 0

    lax.fori_loop(0, HCH, _scat, 0, unroll=False)
    plsc.subcore_barrier()

    def _gath(j, _):
        pltpu.sync_copy(shared_cnt.at[gid2d.at[cid * 4 + j]], cnt_v.at[j])
        return 0

    def _recip(k, _):
        c16 = cnt_v[k // 8, pl.ds((k % 8) * 16, 16)]
        w_v[pl.ds(k * 16, 16)] = 1.0 / c16
        return 0

    lax.fori_loop(0, 4, _gath, 0, unroll=False)
    lax.fori_loop(0, 32, _recip, 0, unroll=False)
    pltpu.sync_copy(w_v, w_hbm.at[pl.ds(hbase + cid * 512, 512)])


def _sc_weights(targets, subgroup_inf):
    mesh = plsc.VectorSubcoreMesh(core_axis_name="c", subcore_axis_name="s")
    f = functools.partial(
        pl.kernel,
        out_type=jax.ShapeDtypeStruct((N,), jnp.float32),
        mesh=mesh,
        scratch_types=[
            pltpu.VMEM((PERH,), jnp.int32),
            pltpu.VMEM((PERH,), jnp.int32),
            pltpu.VMEM((HCH, 128), jnp.int32),
            pltpu.VMEM((128,), jnp.float32),
            pltpu.VMEM((BPW,), jnp.float32),
            pltpu.VMEM((4, 128), jnp.float32),
            pltpu.VMEM((512,), jnp.float32),
            pltpu.SemaphoreType.DMA,
            pltpu.VMEM_SHARED((GP,), jnp.float32),
        ],
    )(_sc_w_body)
    return f(targets, subgroup_inf)


def _tc_body(x_ref, t_ref, o_ref, m_sc, s_sc, p_sc):
    i = pl.program_id(0)
    x = x_ref[...]
    t = t_ref[...]
    ones = jnp.ones((1, CROWS), jnp.float32)
    cmax = jnp.max(x, axis=0, keepdims=True)

    @pl.when(i == 0)
    def _():
        m_sc[...] = jnp.full((1, N), -jnp.inf, jnp.float32)
        s_sc[...] = jnp.zeros((1, N), jnp.float32)
        p_sc[...] = jnp.zeros((1, N), jnp.float32)

    m_old = m_sc[...]
    m_new = jnp.maximum(m_old, cmax)
    e = jnp.exp(x - m_new)
    row = lax.broadcasted_iota(jnp.int32, (CROWS, N), 0) + i * CROWS
    px = jnp.where(row == t, x, 0.0)
    esum = jnp.dot(ones, e, preferred_element_type=jnp.float32)
    picked = jnp.dot(ones, px, preferred_element_type=jnp.float32)
    m_sc[...] = m_new
    s_sc[...] = s_sc[...] * jnp.exp(m_old - m_new) + esum
    p_sc[...] = p_sc[...] + picked

    @pl.when(i == CGRID - 1)
    def _():
        o_ref[...] = m_sc[...] + jnp.log(s_sc[...]) - p_sc[...]


def _tc_loss(logits_t, targets):
    return pl.pallas_call(
        _tc_body,
        grid=(CGRID,),
        in_specs=[
            pl.BlockSpec((CROWS, N), lambda i: (i, 0)),
            pl.BlockSpec((1, N), lambda i: (0, 0)),
        ],
        out_specs=pl.BlockSpec((1, N), lambda i: (0, 0)),
        out_shape=jax.ShapeDtypeStruct((1, N), jnp.float32),
        scratch_shapes=[
            pltpu.VMEM((1, N), jnp.float32),
            pltpu.VMEM((1, N), jnp.float32),
            pltpu.VMEM((1, N), jnp.float32),
        ],
    )(logits_t, targets[None, :])


def _tc_dot_body(l_ref, w_ref, o_ref):
    o_ref[...] = jnp.sum(l_ref[...] * w_ref[...]).reshape(1, 1)


def _tc_dot(loss, w):
    return pl.pallas_call(
        _tc_dot_body,
        in_specs=[
            pl.BlockSpec((1, N), lambda: (0, 0)),
            pl.BlockSpec((1, N), lambda: (0, 0)),
        ],
        out_specs=pl.BlockSpec((1, 1), lambda: (0, 0)),
        out_shape=jax.ShapeDtypeStruct((1, 1), jnp.float32),
    )(loss, w[None, :])


def kernel(logits, targets, subgroup_inf):
    w = _sc_weights(targets, subgroup_inf)
    loss = _tc_loss(logits.T, targets)
    tot = _tc_dot(loss, w)
    return tot.reshape(())


# speedup vs baseline: 1.1327x; 1.0883x over previous
import functools

import jax
import jax.numpy as jnp
from jax import lax
from jax.experimental import pallas as pl
from jax.experimental.pallas import tpu as pltpu
from jax.experimental.pallas import tpu_sc as plsc

N = 16384
C = 1000
S = 8
GP = 8192

COLS = 2048
GRID = N // COLS

NSUB = 16
PERH = N // NSUB
BPW = GP // NSUB
HCH = PERH // 128


def _sc_w_body(t_hbm, s_hbm, w_hbm,
               t_v, s_v, gid2d, ones_v, zeros_v, cnt_v, w_v, sem,
               shared_cnt):
    cid = lax.axis_index("c")
    sid = lax.axis_index("s")

    one16 = jnp.full((16,), 1.0, jnp.float32)
    zero16 = jnp.zeros((16,), jnp.float32)

    hbase = sid * PERH
    pltpu.sync_copy(t_hbm.at[pl.ds(hbase, PERH)], t_v)
    pltpu.sync_copy(s_hbm.at[pl.ds(hbase, PERH)], s_v)

    def _fill_ones(i, _):
        ones_v[pl.ds(i * 16, 16)] = one16
        return 0

    def _fill_zeros(i, _):
        zeros_v[pl.ds(i * 16, 16)] = zero16
        return 0

    def _gid(k, _):
        t16 = t_v[pl.ds(k * 16, 16)]
        s16 = s_v[pl.ds(k * 16, 16)]
        gid2d[k // 8, pl.ds((k % 8) * 16, 16)] = t16 * S + s16
        return 0

    lax.fori_loop(0, 8, _fill_ones, 0, unroll=False)
    lax.fori_loop(0, BPW // 16, _fill_zeros, 0, unroll=False)
    pltpu.sync_copy(zeros_v, shared_cnt.at[pl.ds(sid * BPW, BPW)])
    lax.fori_loop(0, PERH // 16, _gid, 0, unroll=False)

    plsc.subcore_barrier()

    def _scat(j, _):
        pltpu.sync_copy(ones_v, shared_cnt.at[gid2d.at[j]], add=True)
        return 0

    lax.fori_loop(0, HCH, _scat, 0, unroll=False)
    plsc.subcore_barrier()

    del cid

    def _gath(j, _):
        pltpu.sync_copy(shared_cnt.at[gid2d.at[j]], cnt_v.at[j])
        return 0

    def _recip(k, _):
        c16 = cnt_v[k // 8, pl.ds((k % 8) * 16, 16)]
        w_v[pl.ds(k * 16, 16)] = 1.0 / c16
        return 0

    lax.fori_loop(0, HCH, _gath, 0, unroll=False)
    lax.fori_loop(0, PERH // 16, _recip, 0, unroll=False)
    pltpu.sync_copy(w_v, w_hbm.at[pl.ds(hbase, PERH)])


def _sc_weights(targets, subgroup_inf):
    mesh = plsc.VectorSubcoreMesh(core_axis_name="c", subcore_axis_name="s",
                                  num_cores=1)
    f = functools.partial(
        pl.kernel,
        out_type=jax.ShapeDtypeStruct((N,), jnp.float32),
        mesh=mesh,
        scratch_types=[
            pltpu.VMEM((PERH,), jnp.int32),
            pltpu.VMEM((PERH,), jnp.int32),
            pltpu.VMEM((HCH, 128), jnp.int32),
            pltpu.VMEM((128,), jnp.float32),
            pltpu.VMEM((BPW,), jnp.float32),
            pltpu.VMEM((HCH, 128), jnp.float32),
            pltpu.VMEM((PERH,), jnp.float32),
            pltpu.SemaphoreType.DMA,
            pltpu.VMEM_SHARED((GP,), jnp.float32),
        ],
    )(_sc_w_body)
    return f(targets, subgroup_inf)


def _tc_body(x_ref, t_ref, o_ref):
    x = x_ref[...]
    t = t_ref[...]
    ones = jnp.ones((1, C), jnp.float32)
    m = jnp.max(x, axis=0, keepdims=True)
    e = jnp.exp(x - m)
    row = lax.broadcasted_iota(jnp.int32, (C, COLS), 0)
    px = jnp.where(row == t, x, 0.0)
    ssum = jnp.dot(ones, e, preferred_element_type=jnp.float32)
    picked = jnp.dot(ones, px, preferred_element_type=jnp.float32)
    o_ref[...] = m + jnp.log(ssum) - picked


def _tc_loss(logits_t, targets):
    return pl.pallas_call(
        _tc_body,
        grid=(GRID,),
        in_specs=[
            pl.BlockSpec((C, COLS), lambda i: (0, i)),
            pl.BlockSpec((1, COLS), lambda i: (0, i)),
        ],
        out_specs=pl.BlockSpec((1, COLS), lambda i: (0, i)),
        out_shape=jax.ShapeDtypeStruct((1, N), jnp.float32),
    )(logits_t, targets[None, :])


def _tc_dot_body(l_ref, w_ref, o_ref):
    o_ref[...] = jnp.sum(l_ref[...] * w_ref[...]).reshape(1, 1)


def _tc_dot(loss, w):
    return pl.pallas_call(
        _tc_dot_body,
        in_specs=[
            pl.BlockSpec((1, N), lambda: (0, 0)),
            pl.BlockSpec((1, N), lambda: (0, 0)),
        ],
        out_specs=pl.BlockSpec((1, 1), lambda: (0, 0)),
        out_shape=jax.ShapeDtypeStruct((1, 1), jnp.float32),
    )(loss, w[None, :])


def kernel(logits, targets, subgroup_inf):
    w = _sc_weights(targets, subgroup_inf)
    loss = _tc_loss(logits.T, targets)
    tot = _tc_dot(loss, w)
    return tot.reshape(())
